# Initial kernel scaffold; baseline (speedup 1.0000x reference)
import jax
import jax.numpy as jnp
from jax.experimental import pallas as pl


def kernel(input_ids, attention_mask, W, gamma, beta):
    raise NotImplementedError("write your pallas kernel here")


# speedup vs baseline: 1.3304x; 1.3304x over previous
import functools

import jax
import jax.numpy as jnp
from jax import lax
from jax.experimental import pallas as pl
from jax.experimental.pallas import tpu as pltpu
from jax.experimental.pallas import tpu_sc as plsc

HIDDEN = 1024
TROWS = 40
MASK_ID = 32
ZERO_ROW = 33
LN_EPS = 1e-12

_INFO = plsc.get_sparse_core_info()
NC, NS = _INFO.num_cores, _INFO.num_subcores
NW = NC * NS
CHUNK = 32


def _prep_body(w_ref, g_ref, b_ref, ids_ref, mask_ref, t_ref, idx_ref):
    w = w_ref[...]
    mu = jnp.mean(w, axis=1, keepdims=True)
    var = jnp.mean((w - mu) ** 2, axis=1, keepdims=True)
    normed = (w - mu) * lax.rsqrt(var + LN_EPS) * g_ref[...] + b_ref[...]
    r = lax.broadcasted_iota(jnp.int32, (TROWS, HIDDEN), 0)
    t = jnp.where(r == MASK_ID, b_ref[...], normed)
    t_ref[...] = jnp.where(r >= ZERO_ROW, 0.0, t)
    idx_ref[...] = jnp.where(mask_ref[...] != 0.0, ids_ref[...], ZERO_ROW)


def _make_gather(total):
    b_per_w = total // NW
    nchunk = b_per_w // CHUNK
    mesh = plsc.VectorSubcoreMesh(core_axis_name="c", subcore_axis_name="s")

    @functools.partial(
        pl.kernel,
        mesh=mesh,
        out_type=jax.ShapeDtypeStruct((total, HIDDEN), jnp.float32),
        scratch_types=[
            pltpu.VMEM((b_per_w,), jnp.int32),
            pltpu.VMEM((CHUNK, HIDDEN), jnp.float32),
            pltpu.VMEM((CHUNK, HIDDEN), jnp.float32),
            pltpu.SemaphoreType.DMA,
            pltpu.SemaphoreType.DMA,
            pltpu.SemaphoreType.DMA,
            pltpu.SemaphoreType.DMA,
        ],
    )
    def gather(t_hbm, idx_hbm, out_hbm, idx_v, rows0, rows1, g0, g1, s0, s1):
        wid = lax.axis_index("s") * NC + lax.axis_index("c")
        base = wid * b_per_w
        pltpu.sync_copy(idx_hbm.at[pl.ds(base, b_per_w)], idx_v)

        rows = (rows0, rows1)
        gsem = (g0, g1)
        ssem = (s0, s1)

        def g_copy(k, b):
            return pltpu.make_async_copy(
                t_hbm.at[idx_v.at[pl.ds(k * CHUNK, CHUNK)]], rows[b], gsem[b]
            )

        def s_copy(k, b):
            return pltpu.make_async_copy(
                rows[b], out_hbm.at[pl.ds(base + k * CHUNK, CHUNK)], ssem[b]
            )

        g_copy(0, 0).start()

        def body(i, _):
            for b in range(2):
                k = i * 2 + b
                nb = 1 - b

                @pl.when(k + 1 < nchunk)
                def _():
                    @pl.when(k >= 1)
                    def _():
                        s_copy(k - 1, nb).wait()

                    g_copy(k + 1, nb).start()

                g_copy(k, b).wait()
                s_copy(k, b).start()
            return 0

        lax.fori_loop(0, nchunk // 2, body, 0)
        s_copy(nchunk - 2, 0).wait()
        s_copy(nchunk - 1, 1).wait()

    return gather


def kernel(input_ids, attention_mask, W, gamma, beta):
    B, S = input_ids.shape
    total = B * S
    ids32 = input_ids.astype(jnp.int32)
    w_pad = jnp.zeros((TROWS, HIDDEN), jnp.float32).at[: W.shape[0]].set(W)

    table, idx = pl.pallas_call(
        _prep_body,
        out_shape=(
            jax.ShapeDtypeStruct((TROWS, HIDDEN), jnp.float32),
            jax.ShapeDtypeStruct((B, S), jnp.int32),
        ),
    )(w_pad, gamma.reshape(1, HIDDEN), beta.reshape(1, HIDDEN), ids32,
      attention_mask)

    out = _make_gather(total)(table, idx.reshape(total))
    return out.reshape(B, S, HIDDEN)


# speedup vs baseline: 1.3331x; 1.0021x over previous
import functools

import jax
import jax.numpy as jnp
from jax import lax
from jax.experimental import pallas as pl
from jax.experimental.pallas import tpu as pltpu
from jax.experimental.pallas import tpu_sc as plsc

HIDDEN = 1024
TROWS = 40
MASK_ID = 32
ZERO_ROW = 33
LN_EPS = 1e-12

_INFO = plsc.get_sparse_core_info()
NC, NS = _INFO.num_cores, _INFO.num_subcores
NW = NC * NS
CHUNK = 16
NBUF = 4


def _prep_body(w_ref, g_ref, b_ref, ids_ref, mask_ref, t_ref, idx_ref):
    w = w_ref[...]
    mu = jnp.mean(w, axis=1, keepdims=True)
    var = jnp.mean((w - mu) ** 2, axis=1, keepdims=True)
    normed = (w - mu) * lax.rsqrt(var + LN_EPS) * g_ref[...] + b_ref[...]
    r = lax.broadcasted_iota(jnp.int32, (TROWS, HIDDEN), 0)
    t = jnp.where(r == MASK_ID, b_ref[...], normed)
    t_ref[...] = jnp.where(r >= ZERO_ROW, 0.0, t)
    idx_ref[...] = jnp.where(mask_ref[...] != 0.0, ids_ref[...], ZERO_ROW)


def _make_gather(total):
    b_per_w = total // NW
    nchunk = b_per_w // CHUNK
    mesh = plsc.VectorSubcoreMesh(core_axis_name="c", subcore_axis_name="s")

    @functools.partial(
        pl.kernel,
        mesh=mesh,
        out_type=jax.ShapeDtypeStruct((total, HIDDEN), jnp.float32),
        scratch_types=(
            [pltpu.VMEM((b_per_w,), jnp.int32)]
            + [pltpu.VMEM((CHUNK, HIDDEN), jnp.float32) for _ in range(NBUF)]
            + [pltpu.SemaphoreType.DMA for _ in range(2 * NBUF)]
        ),
    )
    def gather(t_hbm, idx_hbm, out_hbm, idx_v, *bufs):
        rows = bufs[:NBUF]
        gsem = bufs[NBUF : 2 * NBUF]
        ssem = bufs[2 * NBUF :]
        wid = lax.axis_index("s") * NC + lax.axis_index("c")
        base = wid * b_per_w
        pltpu.sync_copy(idx_hbm.at[pl.ds(base, b_per_w)], idx_v)

        def g_copy(k, b):
            return pltpu.make_async_copy(
                t_hbm.at[idx_v.at[pl.ds(k * CHUNK, CHUNK)]], rows[b], gsem[b]
            )

        def s_copy(k, b):
            return pltpu.make_async_copy(
                rows[b], out_hbm.at[pl.ds(base + k * CHUNK, CHUNK)], ssem[b]
            )

        for j in range(NBUF - 1):
            g_copy(j, j).start()

        def body(i, _):
            for b in range(NBUF):
                k = i * NBUF + b
                pb = (b - 1) % NBUF

                @pl.when(k + NBUF - 1 < nchunk)
                def _():
                    @pl.when(k >= 1)
                    def _():
                        s_copy(k - 1, pb).wait()

                    g_copy(k + NBUF - 1, pb).start()

                g_copy(k, b).wait()
                s_copy(k, b).start()
            return 0

        lax.fori_loop(0, nchunk // NBUF, body, 0)
        for j in range(NBUF):
            k = nchunk - NBUF + j
            s_copy(k, k % NBUF).wait()

    return gather


def kernel(input_ids, attention_mask, W, gamma, beta):
    B, S = input_ids.shape
    total = B * S
    ids32 = input_ids.astype(jnp.int32)
    w_pad = jnp.zeros((TROWS, HIDDEN), jnp.float32).at[: W.shape[0]].set(W)

    table, idx = pl.pallas_call(
        _prep_body,
        out_shape=(
            jax.ShapeDtypeStruct((TROWS, HIDDEN), jnp.float32),
            jax.ShapeDtypeStruct((B, S), jnp.int32),
        ),
    )(w_pad, gamma.reshape(1, HIDDEN), beta.reshape(1, HIDDEN), ids32,
      attention_mask)

    out = _make_gather(total)(table, idx.reshape(total))
    return out.reshape(B, S, HIDDEN)
